# baseline (speedup 1.0000x reference)
import functools

import jax
import jax.numpy as jnp
from jax import lax
from jax.experimental import pallas as pl
from jax.experimental.pallas import tpu as pltpu
from jax.experimental.pallas import tpu_sc as plsc

N = 10000
E = 320000
D = 128
H = 256
HH = 128
L = 128
G = 64

NC = 2
NS = 16
LN = 16

EPT = E // NS
B = 80
NB = EPT // B
RPT = 624
RTAIL = N - NS * RPT
ZR = 78
NBLK = 5
BN = N // NBLK
BNP = 2048
EBLK = 80
BE = E // EBLK


def _proj_body(x_ref, wi_ref, wj_ref, pi_ref, pj_ref):
  xb = x_ref[...]
  pi = jnp.dot(xb, wi_ref[...], preferred_element_type=jnp.float32)
  pj = jnp.dot(xb, wj_ref[...], preferred_element_type=jnp.float32)
  pi_ref[0] = pi[:, :HH]
  pi_ref[1] = pi[:, HH:]
  pj_ref[0] = pj[:, :HH]
  pj_ref[1] = pj[:, HH:]


def _tc_proj(F, Wi, Wj):
  Dn = F.shape[1]
  return pl.pallas_call(
      _proj_body,
      grid=(NBLK,),
      in_specs=[
          pl.BlockSpec((BN, Dn), lambda i: (i, 0)),
          pl.BlockSpec((Dn, H), lambda i: (0, 0)),
          pl.BlockSpec((Dn, H), lambda i: (0, 0)),
      ],
      out_specs=[
          pl.BlockSpec((NC, BN, HH), lambda i: (0, i, 0)),
          pl.BlockSpec((NC, BN, HH), lambda i: (0, i, 0)),
      ],
      out_shape=[
          jax.ShapeDtypeStruct((NC, N, HH), jnp.float32),
          jax.ShapeDtypeStruct((NC, N, HH), jnp.float32),
      ],
  )(F, Wi, Wj)


def _ebias_body(ea_ref, w_ref, b_ref, t_ref):
  t = jnp.dot(ea_ref[...], w_ref[...], preferred_element_type=jnp.float32)
  t = t + b_ref[...]
  t_ref[0] = t[:, :HH]
  t_ref[1] = t[:, HH:]


def _tc_ebias(ea8, Wae8, ba):
  return pl.pallas_call(
      _ebias_body,
      grid=(EBLK,),
      in_specs=[
          pl.BlockSpec((BE, 8), lambda i: (i, 0)),
          pl.BlockSpec((8, H), lambda i: (0, 0)),
          pl.BlockSpec((1, H), lambda i: (0, 0)),
      ],
      out_specs=[pl.BlockSpec((NC, BE, HH), lambda i: (0, i, 0))],
      out_shape=[jax.ShapeDtypeStruct((NC, E, HH), jnp.float32)],
  )(ea8, Wae8, ba)[0]


def _stats_body(st_ref, g_ref, be_ref, sc_ref, sh_ref):
  st = st_ref[...]
  sums = jnp.sum(st, axis=1) * (1.0 / E)
  m = sums[:NC]
  msq = sums[NC:]
  v = msq - m * m
  scale = g_ref[...] * lax.rsqrt(v + 1e-5)
  sc_ref[...] = scale
  sh_ref[...] = be_ref[...] - m * scale


def _tc_stats(stats, g, be):
  return pl.pallas_call(
      _stats_body,
      out_shape=[
          jax.ShapeDtypeStruct((NC, HH), jnp.float32),
          jax.ShapeDtypeStruct((NC, HH), jnp.float32),
      ],
  )(stats.reshape(2 * NC, NS, HH), g.reshape(NC, HH), be.reshape(NC, HH))


def _update_body(s_ref, cnt_ref, wb_ref, bb_ref, *out_refs):
  cnt = cnt_ref[0][:, 0:1] + cnt_ref[1][:, 0:1]
  inv = 1.0 / jnp.maximum(cnt, 1.0)
  a = jnp.concatenate([s_ref[0], s_ref[1]], axis=1) * inv
  hn = jnp.dot(a, wb_ref[...], preferred_element_type=jnp.float32)
  hn = hn + bb_ref[...]
  out_refs[0][...] = hn
  if len(out_refs) > 1:
    w2i = out_refs[-2]
    w2j = out_refs[-1]
    pi = jnp.dot(hn, w2i[...], preferred_element_type=jnp.float32)
    pj = jnp.dot(hn, w2j[...], preferred_element_type=jnp.float32)
    out_refs[1][0] = pi[:, :HH]
    out_refs[1][1] = pi[:, HH:]
    out_refs[2][0] = pj[:, :HH]
    out_refs[2][1] = pj[:, HH:]


def _tc_update(sH, cnt_parts, Wb, bb, W2i=None, W2j=None):
  in_specs = [
      pl.BlockSpec((NC, BN, HH), lambda i: (0, i, 0)),
      pl.BlockSpec((NC, BN, HH), lambda i: (0, i, 0)),
      pl.BlockSpec((H, H), lambda i: (0, 0)),
      pl.BlockSpec((1, H), lambda i: (0, 0)),
  ]
  out_specs = [pl.BlockSpec((BN, H), lambda i: (i, 0))]
  out_shape = [jax.ShapeDtypeStruct((N, H), jnp.float32)]
  args = [sH, cnt_parts, Wb, bb]
  if W2i is not None:
    in_specs += [pl.BlockSpec((H, H), lambda i: (0, 0))] * 2
    out_specs += [pl.BlockSpec((NC, BN, HH), lambda i: (0, i, 0))] * 2
    out_shape += [jax.ShapeDtypeStruct((NC, N, HH), jnp.float32)] * 2
    args += [W2i, W2j]

  def body(s_ref, cnt_ref, wb_ref, bb_ref, *rest):
    if W2i is not None:
      w2i_ref, w2j_ref = rest[:2]
      outs = rest[2:] + (w2i_ref, w2j_ref)
    else:
      outs = rest
    _update_body(s_ref, cnt_ref, wb_ref, bb_ref, *outs)

  return pl.pallas_call(
      body, grid=(NBLK,), in_specs=in_specs, out_specs=out_specs,
      out_shape=out_shape)(*args)


def _pool_body(b_ref, h_ref, p_ref):
  i = pl.program_id(0)

  @pl.when(i == 0)
  def _():
    p_ref[...] = jnp.zeros_like(p_ref)

  bb = b_ref[0]
  gids = lax.broadcasted_iota(jnp.int32, (G, BN), 0)
  onehot = (bb == gids).astype(jnp.float32)
  p_ref[...] += jnp.dot(onehot, h_ref[...],
                        preferred_element_type=jnp.float32)


def _tc_pool(h2, batch3):
  return pl.pallas_call(
      _pool_body,
      grid=(NBLK,),
      in_specs=[
          pl.BlockSpec((1, 1, BN), lambda i: (i, 0, 0)),
          pl.BlockSpec((BN, H), lambda i: (i, 0)),
      ],
      out_specs=pl.BlockSpec((G, H), lambda i: (0, 0)),
      out_shape=jax.ShapeDtypeStruct((G, H), jnp.float32),
  )(batch3, h2)


def _head_body(p_ref, gp_ref, bp_ref, wmu_ref, bmu_ref, wlv_ref, blv_ref,
               mu_ref, lv_ref):
  p = p_ref[...]
  m = jnp.mean(p, axis=0, keepdims=True)
  v = jnp.mean(p * p, axis=0, keepdims=True) - m * m
  pb = (p - m) * lax.rsqrt(v + 1e-5) * gp_ref[...] + bp_ref[...]
  mu_ref[...] = jnp.dot(pb, wmu_ref[...],
                        preferred_element_type=jnp.float32) + bmu_ref[...]
  lv_ref[...] = jnp.dot(pb, wlv_ref[...],
                        preferred_element_type=jnp.float32) + blv_ref[...]


def _tc_head(pooled, gp, bp, Wmu, bmu, Wlv, blv):
  return pl.pallas_call(
      _head_body,
      out_shape=[
          jax.ShapeDtypeStruct((G, L), jnp.float32),
          jax.ShapeDtypeStruct((G, L), jnp.float32),
      ],
  )(pooled, gp.reshape(1, H), bp.reshape(1, H), Wmu, bmu.reshape(1, L),
    Wlv, blv.reshape(1, L))


_MESH = plsc.VectorSubcoreMesh(core_axis_name="c", subcore_axis_name="s")


@functools.partial(
    pl.kernel, mesh=_MESH,
    out_type=[
        jax.ShapeDtypeStruct((NC, E, HH), jnp.float32),
        jax.ShapeDtypeStruct((2, NC, NS * HH), jnp.float32),
    ],
    scratch_types=[
        pltpu.VMEM((2, B), jnp.int32),
        pltpu.VMEM((2, B), jnp.int32),
        pltpu.VMEM((2, B, HH), jnp.float32),
        pltpu.VMEM((2, B, HH), jnp.float32),
        pltpu.VMEM((2, B, HH), jnp.float32),
        pltpu.VMEM((HH,), jnp.float32),
        pltpu.VMEM((HH,), jnp.float32),
        pltpu.SemaphoreType.DMA,
        pltpu.SemaphoreType.DMA,
        pltpu.SemaphoreType.DMA,
        pltpu.SemaphoreType.DMA,
        pltpu.SemaphoreType.DMA,
        pltpu.SemaphoreType.DMA,
    ])
def _sc_pass_a(pi_h, pj_h, t_h, dst_h, src_h, h_out, st_out,
               idx_d, idx_s, gi, gj, hb, asum, asq,
               isem0, isem1, gsem0, gsem1, wsem0, wsem1):
  c = lax.axis_index("c")
  s = lax.axis_index("s")
  isem = (isem0, isem1)
  gsem = (gsem0, gsem1)
  wsem = (wsem0, wsem1)
  for q in range(HH // LN):
    asum[pl.ds(q * LN, LN)] = jnp.zeros((LN,), jnp.float32)
    asq[pl.ds(q * LN, LN)] = jnp.zeros((LN,), jnp.float32)

  def issue_idx(b, k):
    eb = s * EPT + b * B
    pltpu.async_copy(dst_h.at[pl.ds(eb, B)], idx_d.at[k], isem[k])
    pltpu.async_copy(src_h.at[pl.ds(eb, B)], idx_s.at[k], isem[k])

  def wait_idx(k):
    pltpu.make_async_copy(dst_h.at[pl.ds(0, B)], idx_d.at[k], isem[k]).wait()
    pltpu.make_async_copy(src_h.at[pl.ds(0, B)], idx_s.at[k], isem[k]).wait()

  def issue_gather(b, k):
    eb = s * EPT + b * B
    pltpu.async_copy(pi_h.at[c].at[idx_d.at[k]], gi.at[k], gsem[k])
    pltpu.async_copy(pj_h.at[c].at[idx_s.at[k]], gj.at[k], gsem[k])
    pltpu.async_copy(t_h.at[c].at[pl.ds(eb, B)], hb.at[k], gsem[k])

  def wait_gather(k):
    dummy = t_h.at[c].at[pl.ds(0, B)]
    pltpu.make_async_copy(dummy, gi.at[k], gsem[k]).wait()
    pltpu.make_async_copy(dummy, gj.at[k], gsem[k]).wait()
    pltpu.make_async_copy(dummy, hb.at[k], gsem[k]).wait()

  def wait_wb(k):
    pltpu.make_async_copy(t_h.at[c].at[pl.ds(0, B)], hb.at[k],
                          wsem[k]).wait()

  pltpu.sync_copy(dst_h.at[pl.ds(s * EPT, B)], idx_d.at[0])
  pltpu.sync_copy(src_h.at[pl.ds(s * EPT, B)], idx_s.at[0])
  issue_gather(0, 0)
  issue_idx(1, 1)

  def stage(b, k, nk):
    @pl.when(b + 1 < NB)
    def _():
      wait_idx(nk)

      @pl.when(b >= 1)
      def _():
        wait_wb(nk)

      issue_gather(b + 1, nk)

    wait_gather(k)

    @pl.when(b + 2 < NB)
    def _():
      issue_idx(b + 2, k)

    for q in range(HH // LN):
      sl = pl.ds(q * LN, LN)

      def row_body(r4, car):
        su, sq = car
        for u in range(8):
          r = 8 * r4 + u
          v = gi[k, r, sl] + gj[k, r, sl] + hb[k, r, sl]
          hb[k, r, sl] = v
          su = su + v
          sq = sq + v * v
        return su, sq

      su, sq = lax.fori_loop(0, B // 8, row_body, (asum[sl], asq[sl]))
      asum[sl] = su
      asq[sl] = sq

    eb = s * EPT + b * B
    pltpu.async_copy(hb.at[k], h_out.at[c].at[pl.ds(eb, B)], wsem[k])

  def outer(g, _):
    stage(2 * g, 0, 1)
    stage(2 * g + 1, 1, 0)
    return 0

  lax.fori_loop(0, NB // 2, outer, 0)
  wait_wb(0)
  wait_wb(1)
  pltpu.sync_copy(asum, st_out.at[0].at[c].at[pl.ds(s * HH, HH)])
  pltpu.sync_copy(asq, st_out.at[1].at[c].at[pl.ds(s * HH, HH)])


@functools.partial(
    pl.kernel, mesh=_MESH,
    out_type=jax.ShapeDtypeStruct((NC, N, HH), jnp.float32),
    scratch_types=[
        pltpu.VMEM((2, B), jnp.int32),
        pltpu.VMEM((2, B, HH), jnp.float32),
        pltpu.VMEM((ZR, HH), jnp.float32),
        pltpu.VMEM((HH,), jnp.float32),
        pltpu.VMEM((HH,), jnp.float32),
        pltpu.VMEM_SHARED((N, HH), jnp.float32),
        pltpu.SemaphoreType.DMA,
        pltpu.SemaphoreType.DMA,
        pltpu.SemaphoreType.DMA,
        pltpu.SemaphoreType.DMA,
    ])
def _sc_pass_b(h_h, dst_h, scale_h, shift_h, s_out,
               idx_d, hb, zbuf, sc_v, sh_v, acc,
               hsem0, hsem1, ssem0, ssem1):
  c = lax.axis_index("c")
  s = lax.axis_index("s")
  hsem = (hsem0, hsem1)
  ssem = (ssem0, ssem1)

  def zrow(r, _):
    for q in range(HH // LN):
      zbuf[r, pl.ds(q * LN, LN)] = jnp.zeros((LN,), jnp.float32)
    return 0

  lax.fori_loop(0, ZR, zrow, 0)
  for j in range(RPT // ZR):
    pltpu.sync_copy(zbuf, acc.at[pl.ds(s * RPT + j * ZR, ZR)])

  @pl.when(s == NS - 1)
  def _():
    pltpu.sync_copy(zbuf.at[pl.ds(0, RTAIL)], acc.at[pl.ds(NS * RPT, RTAIL)])

  pltpu.sync_copy(scale_h.at[c], sc_v)
  pltpu.sync_copy(shift_h.at[c], sh_v)
  plsc.subcore_barrier()

  def issue_load(b, k):
    eb = s * EPT + b * B
    pltpu.async_copy(dst_h.at[pl.ds(eb, B)], idx_d.at[k], hsem[k])
    pltpu.async_copy(h_h.at[c].at[pl.ds(eb, B)], hb.at[k], hsem[k])

  def wait_load(k):
    pltpu.make_async_copy(dst_h.at[pl.ds(0, B)], idx_d.at[k],
                          hsem[k]).wait()
    pltpu.make_async_copy(h_h.at[c].at[pl.ds(0, B)], hb.at[k],
                          hsem[k]).wait()

  def wait_scatter(k):
    pltpu.make_async_copy(h_h.at[c].at[pl.ds(0, B)], hb.at[k],
                          ssem[k]).wait()

  issue_load(0, 0)

  def stage(b, k, nk):
    @pl.when(b + 1 < NB)
    def _():
      @pl.when(b >= 1)
      def _():
        wait_scatter(nk)

      issue_load(b + 1, nk)

    wait_load(k)

    for q in range(HH // LN):
      sl = pl.ds(q * LN, LN)

      def row_body(r4, _):
        for u in range(8):
          r = 8 * r4 + u
          v = hb[k, r, sl] * sc_v[sl] + sh_v[sl]
          hb[k, r, sl] = jnp.maximum(v, 0.2 * v)
        return 0

      lax.fori_loop(0, B // 8, row_body, 0)

    pltpu.async_copy(hb.at[k], acc.at[idx_d.at[k]], ssem[k], add=True)

  def outer(g, _):
    stage(2 * g, 0, 1)
    stage(2 * g + 1, 1, 0)
    return 0

  lax.fori_loop(0, NB // 2, outer, 0)
  wait_scatter(0)
  wait_scatter(1)
  plsc.subcore_barrier()
  pltpu.sync_copy(acc.at[pl.ds(s * RPT, RPT)],
                  s_out.at[c].at[pl.ds(s * RPT, RPT)])

  @pl.when(s == NS - 1)
  def _():
    pltpu.sync_copy(acc.at[pl.ds(NS * RPT, RTAIL)],
                    s_out.at[c].at[pl.ds(NS * RPT, RTAIL)])


EPC = E // NC
EPCT = EPC // NS
NBC = EPCT // B


@functools.partial(
    pl.kernel, mesh=_MESH,
    out_type=jax.ShapeDtypeStruct((NC, N, HH), jnp.float32),
    scratch_types=[
        pltpu.VMEM((B,), jnp.int32),
        pltpu.VMEM((B, HH), jnp.float32),
        pltpu.VMEM((ZR, HH), jnp.float32),
        pltpu.VMEM_SHARED((N, HH), jnp.float32),
    ])
def _sc_count(dst_h, cnt_out, idx_d, ones_b, zbuf, cnt_sh):
  c = lax.axis_index("c")
  s = lax.axis_index("s")

  def zrow(r, _):
    for q in range(HH // LN):
      zbuf[r, pl.ds(q * LN, LN)] = jnp.zeros((LN,), jnp.float32)
    return 0

  lax.fori_loop(0, ZR, zrow, 0)
  for j in range(RPT // ZR):
    pltpu.sync_copy(zbuf, cnt_sh.at[pl.ds(s * RPT + j * ZR, ZR)])

  @pl.when(s == NS - 1)
  def _():
    pltpu.sync_copy(zbuf.at[pl.ds(0, RTAIL)],
                    cnt_sh.at[pl.ds(NS * RPT, RTAIL)])

  def orow(r, _):
    for q in range(HH // LN):
      ones_b[r, pl.ds(q * LN, LN)] = jnp.full((LN,), 1.0, jnp.float32)
    return 0

  lax.fori_loop(0, B, orow, 0)
  plsc.subcore_barrier()

  def batch_body(b, _):
    eb = c * EPC + s * EPCT + b * B
    pltpu.sync_copy(dst_h.at[pl.ds(eb, B)], idx_d)
    pltpu.sync_copy(ones_b, cnt_sh.at[idx_d], add=True)
    return 0

  lax.fori_loop(0, NBC, batch_body, 0)
  plsc.subcore_barrier()
  pltpu.sync_copy(cnt_sh.at[pl.ds(s * RPT, RPT)],
                  cnt_out.at[c].at[pl.ds(s * RPT, RPT)])

  @pl.when(s == NS - 1)
  def _():
    pltpu.sync_copy(cnt_sh.at[pl.ds(NS * RPT, RTAIL)],
                    cnt_out.at[c].at[pl.ds(NS * RPT, RTAIL)])


def _layer_sc(PiT, PjT, T, dst, src, g, be):
  h_e, stats = _sc_pass_a(PiT, PjT, T, dst, src)
  scale, shift = _tc_stats(stats, g, be)
  return _sc_pass_b(h_e, dst, scale, shift)


def _layer_sc_cnt(PiT, PjT, T, dst, src, g, be):
  return _layer_sc(PiT, PjT, T, dst, src, g, be)


def kernel(x, edge_index, edge_attr, batch,
           W1a, b1a, g1, be1, W1b, b1b,
           W2a, b2a, g2, be2, W2b, b2b,
           gp, bp, Wmu, bmu, Wlv, blv):
  src = edge_index[0]
  dst = edge_index[1]
  ea8 = jnp.concatenate(
      [edge_attr, jnp.zeros((E, 4), jnp.float32)], axis=1)
  Wae1 = jnp.concatenate([W1a[2 * D:], jnp.zeros((4, H), jnp.float32)], 0)
  Wae2 = jnp.concatenate([W2a[2 * H:], jnp.zeros((4, H), jnp.float32)], 0)

  PiT1, PjT1 = _tc_proj(x, W1a[:D], W1a[D:2 * D])
  T1 = _tc_ebias(ea8, Wae1, b1a.reshape(1, H))
  cnt_tab = _sc_count(dst)
  s1 = _layer_sc(PiT1, PjT1, T1, dst, src, g1, be1)
  h_nodes, PiT2, PjT2 = _tc_update(s1, cnt_tab, W1b, b1b.reshape(1, H),
                                   W2a[:H], W2a[H:2 * H])
  T2 = _tc_ebias(ea8, Wae2, b2a.reshape(1, H))
  s2 = _layer_sc(PiT2, PjT2, T2, dst, src, g2, be2)
  h2 = _tc_update(s2, cnt_tab, W2b, b2b.reshape(1, H))[0]

  pooled = _tc_pool(h2, batch.reshape(NBLK, 1, BN))
  mu, lv = _tc_head(pooled, gp, bp, Wmu, bmu, Wlv, blv)
  return mu, lv, h2


# speedup vs baseline: 1.0307x; 1.0307x over previous
import functools

import jax
import jax.numpy as jnp
from jax import lax
from jax.experimental import pallas as pl
from jax.experimental.pallas import tpu as pltpu
from jax.experimental.pallas import tpu_sc as plsc

N = 10000
E = 320000
D = 128
H = 256
HH = 128
L = 128
G = 64

NC = 2
NS = 16
LN = 16

EPT = E // NS
B = 80
NB = EPT // B
RPT = 624
RTAIL = N - NS * RPT
ZR = 78
NBLK = 5
BN = N // NBLK
BNP = 2048
EBLK = 80
BE = E // EBLK


def _proj_body(x_ref, wi_ref, wj_ref, pi_ref, pj_ref):
  xb = x_ref[...]
  pi = jnp.dot(xb, wi_ref[...], preferred_element_type=jnp.float32)
  pj = jnp.dot(xb, wj_ref[...], preferred_element_type=jnp.float32)
  pi_ref[0] = pi[:, :HH]
  pi_ref[1] = pi[:, HH:]
  pj_ref[0] = pj[:, :HH]
  pj_ref[1] = pj[:, HH:]


def _tc_proj(F, Wi, Wj):
  Dn = F.shape[1]
  return pl.pallas_call(
      _proj_body,
      grid=(NBLK,),
      in_specs=[
          pl.BlockSpec((BN, Dn), lambda i: (i, 0)),
          pl.BlockSpec((Dn, H), lambda i: (0, 0)),
          pl.BlockSpec((Dn, H), lambda i: (0, 0)),
      ],
      out_specs=[
          pl.BlockSpec((NC, BN, HH), lambda i: (0, i, 0)),
          pl.BlockSpec((NC, BN, HH), lambda i: (0, i, 0)),
      ],
      out_shape=[
          jax.ShapeDtypeStruct((NC, N, HH), jnp.float32),
          jax.ShapeDtypeStruct((NC, N, HH), jnp.float32),
      ],
  )(F, Wi, Wj)


def _ebias_body(ea_ref, w_ref, b_ref, t_ref):
  t = jnp.dot(ea_ref[...], w_ref[...], preferred_element_type=jnp.float32)
  t = t + b_ref[...]
  t_ref[0] = t[:, :HH]
  t_ref[1] = t[:, HH:]


def _tc_ebias(ea8, Wae8, ba):
  return pl.pallas_call(
      _ebias_body,
      grid=(EBLK,),
      in_specs=[
          pl.BlockSpec((BE, 8), lambda i: (i, 0)),
          pl.BlockSpec((8, H), lambda i: (0, 0)),
          pl.BlockSpec((1, H), lambda i: (0, 0)),
      ],
      out_specs=[pl.BlockSpec((NC, BE, HH), lambda i: (0, i, 0))],
      out_shape=[jax.ShapeDtypeStruct((NC, E, HH), jnp.float32)],
  )(ea8, Wae8, ba)[0]


def _stats_body(st_ref, g_ref, be_ref, sc_ref, sh_ref):
  st = st_ref[...]
  sums = jnp.sum(st, axis=1) * (1.0 / E)
  m = sums[:NC]
  msq = sums[NC:]
  v = msq - m * m
  scale = g_ref[...] * lax.rsqrt(v + 1e-5)
  sc_ref[...] = scale
  sh_ref[...] = be_ref[...] - m * scale


def _tc_stats(stats, g, be):
  return pl.pallas_call(
      _stats_body,
      out_shape=[
          jax.ShapeDtypeStruct((NC, HH), jnp.float32),
          jax.ShapeDtypeStruct((NC, HH), jnp.float32),
      ],
  )(stats.reshape(2 * NC, NS, HH), g.reshape(NC, HH), be.reshape(NC, HH))


def _update_body(s_ref, cnt_ref, wb_ref, bb_ref, *out_refs):
  cnt = cnt_ref[0][:, 0:1] + cnt_ref[1][:, 0:1]
  inv = 1.0 / jnp.maximum(cnt, 1.0)
  a = jnp.concatenate([s_ref[0], s_ref[1]], axis=1) * inv
  hn = jnp.dot(a, wb_ref[...], preferred_element_type=jnp.float32)
  hn = hn + bb_ref[...]
  out_refs[0][...] = hn
  if len(out_refs) > 1:
    w2i = out_refs[-2]
    w2j = out_refs[-1]
    pi = jnp.dot(hn, w2i[...], preferred_element_type=jnp.float32)
    pj = jnp.dot(hn, w2j[...], preferred_element_type=jnp.float32)
    out_refs[1][0] = pi[:, :HH]
    out_refs[1][1] = pi[:, HH:]
    out_refs[2][0] = pj[:, :HH]
    out_refs[2][1] = pj[:, HH:]


def _tc_update(sH, cnt_parts, Wb, bb, W2i=None, W2j=None):
  in_specs = [
      pl.BlockSpec((NC, BN, HH), lambda i: (0, i, 0)),
      pl.BlockSpec((NC, BN, HH), lambda i: (0, i, 0)),
      pl.BlockSpec((H, H), lambda i: (0, 0)),
      pl.BlockSpec((1, H), lambda i: (0, 0)),
  ]
  out_specs = [pl.BlockSpec((BN, H), lambda i: (i, 0))]
  out_shape = [jax.ShapeDtypeStruct((N, H), jnp.float32)]
  args = [sH, cnt_parts, Wb, bb]
  if W2i is not None:
    in_specs += [pl.BlockSpec((H, H), lambda i: (0, 0))] * 2
    out_specs += [pl.BlockSpec((NC, BN, HH), lambda i: (0, i, 0))] * 2
    out_shape += [jax.ShapeDtypeStruct((NC, N, HH), jnp.float32)] * 2
    args += [W2i, W2j]

  def body(s_ref, cnt_ref, wb_ref, bb_ref, *rest):
    if W2i is not None:
      w2i_ref, w2j_ref = rest[:2]
      outs = rest[2:] + (w2i_ref, w2j_ref)
    else:
      outs = rest
    _update_body(s_ref, cnt_ref, wb_ref, bb_ref, *outs)

  return pl.pallas_call(
      body, grid=(NBLK,), in_specs=in_specs, out_specs=out_specs,
      out_shape=out_shape)(*args)


def _pool_body(b_ref, h_ref, p_ref):
  i = pl.program_id(0)

  @pl.when(i == 0)
  def _():
    p_ref[...] = jnp.zeros_like(p_ref)

  bb = b_ref[0]
  gids = lax.broadcasted_iota(jnp.int32, (G, BN), 0)
  onehot = (bb == gids).astype(jnp.float32)
  p_ref[...] += jnp.dot(onehot, h_ref[...],
                        preferred_element_type=jnp.float32)


def _tc_pool(h2, batch3):
  return pl.pallas_call(
      _pool_body,
      grid=(NBLK,),
      in_specs=[
          pl.BlockSpec((1, 1, BN), lambda i: (i, 0, 0)),
          pl.BlockSpec((BN, H), lambda i: (i, 0)),
      ],
      out_specs=pl.BlockSpec((G, H), lambda i: (0, 0)),
      out_shape=jax.ShapeDtypeStruct((G, H), jnp.float32),
  )(batch3, h2)


def _head_body(p_ref, gp_ref, bp_ref, wmu_ref, bmu_ref, wlv_ref, blv_ref,
               mu_ref, lv_ref):
  p = p_ref[...]
  m = jnp.mean(p, axis=0, keepdims=True)
  v = jnp.mean(p * p, axis=0, keepdims=True) - m * m
  pb = (p - m) * lax.rsqrt(v + 1e-5) * gp_ref[...] + bp_ref[...]
  mu_ref[...] = jnp.dot(pb, wmu_ref[...],
                        preferred_element_type=jnp.float32) + bmu_ref[...]
  lv_ref[...] = jnp.dot(pb, wlv_ref[...],
                        preferred_element_type=jnp.float32) + blv_ref[...]


def _tc_head(pooled, gp, bp, Wmu, bmu, Wlv, blv):
  return pl.pallas_call(
      _head_body,
      out_shape=[
          jax.ShapeDtypeStruct((G, L), jnp.float32),
          jax.ShapeDtypeStruct((G, L), jnp.float32),
      ],
  )(pooled, gp.reshape(1, H), bp.reshape(1, H), Wmu, bmu.reshape(1, L),
    Wlv, blv.reshape(1, L))


_MESH = plsc.VectorSubcoreMesh(core_axis_name="c", subcore_axis_name="s")


@functools.partial(
    pl.kernel, mesh=_MESH,
    out_type=[
        jax.ShapeDtypeStruct((NC, E, HH), jnp.float32),
        jax.ShapeDtypeStruct((2, NC, NS * HH), jnp.float32),
    ],
    scratch_types=[
        pltpu.VMEM((2, B), jnp.int32),
        pltpu.VMEM((2, B), jnp.int32),
        pltpu.VMEM((2, B, HH), jnp.float32),
        pltpu.VMEM((2, B, HH), jnp.float32),
        pltpu.VMEM((2, B, HH), jnp.float32),
        pltpu.VMEM((HH,), jnp.float32),
        pltpu.VMEM((HH,), jnp.float32),
        pltpu.SemaphoreType.DMA,
        pltpu.SemaphoreType.DMA,
        pltpu.SemaphoreType.DMA,
        pltpu.SemaphoreType.DMA,
        pltpu.SemaphoreType.DMA,
        pltpu.SemaphoreType.DMA,
    ])
def _sc_pass_a(pi_h, pj_h, t_h, dst_h, src_h, h_out, st_out,
               idx_d, idx_s, gi, gj, hb, asum, asq,
               isem0, isem1, gsem0, gsem1, wsem0, wsem1):
  c = lax.axis_index("c")
  s = lax.axis_index("s")
  isem = (isem0, isem1)
  gsem = (gsem0, gsem1)
  wsem = (wsem0, wsem1)
  for q in range(HH // LN):
    asum[pl.ds(q * LN, LN)] = jnp.zeros((LN,), jnp.float32)
    asq[pl.ds(q * LN, LN)] = jnp.zeros((LN,), jnp.float32)

  def issue_idx(b, k):
    eb = s * EPT + b * B
    pltpu.async_copy(dst_h.at[pl.ds(eb, B)], idx_d.at[k], isem[k])
    pltpu.async_copy(src_h.at[pl.ds(eb, B)], idx_s.at[k], isem[k])

  def wait_idx(k):
    pltpu.make_async_copy(dst_h.at[pl.ds(0, B)], idx_d.at[k], isem[k]).wait()
    pltpu.make_async_copy(src_h.at[pl.ds(0, B)], idx_s.at[k], isem[k]).wait()

  def issue_gather(b, k):
    eb = s * EPT + b * B
    pltpu.async_copy(pi_h.at[c].at[idx_d.at[k]], gi.at[k], gsem[k])
    pltpu.async_copy(pj_h.at[c].at[idx_s.at[k]], gj.at[k], gsem[k])
    pltpu.async_copy(t_h.at[c].at[pl.ds(eb, B)], hb.at[k], gsem[k])

  def wait_gather(k):
    dummy = t_h.at[c].at[pl.ds(0, B)]
    pltpu.make_async_copy(dummy, gi.at[k], gsem[k]).wait()
    pltpu.make_async_copy(dummy, gj.at[k], gsem[k]).wait()
    pltpu.make_async_copy(dummy, hb.at[k], gsem[k]).wait()

  def wait_wb(k):
    pltpu.make_async_copy(t_h.at[c].at[pl.ds(0, B)], hb.at[k],
                          wsem[k]).wait()

  pltpu.sync_copy(dst_h.at[pl.ds(s * EPT, B)], idx_d.at[0])
  pltpu.sync_copy(src_h.at[pl.ds(s * EPT, B)], idx_s.at[0])
  issue_gather(0, 0)
  issue_idx(1, 1)

  def stage(b, k, nk):
    @pl.when(b + 1 < NB)
    def _():
      wait_idx(nk)

      @pl.when(b >= 1)
      def _():
        wait_wb(nk)

      issue_gather(b + 1, nk)

    wait_gather(k)

    @pl.when(b + 2 < NB)
    def _():
      issue_idx(b + 2, k)

    def row_body(r2, car):
      sus, sqs = car
      nsus, nsqs = [], []
      for q in range(HH // LN):
        sl = pl.ds(q * LN, LN)
        su, sq = sus[q], sqs[q]
        for u in range(2):
          r = 2 * r2 + u
          v = gi[k, r, sl] + gj[k, r, sl] + hb[k, r, sl]
          hb[k, r, sl] = v
          su = su + v
          sq = sq + v * v
        nsus.append(su)
        nsqs.append(sq)
      return tuple(nsus), tuple(nsqs)

    init = (tuple(asum[pl.ds(q * LN, LN)] for q in range(HH // LN)),
            tuple(asq[pl.ds(q * LN, LN)] for q in range(HH // LN)))
    sus, sqs = lax.fori_loop(0, B // 2, row_body, init)
    for q in range(HH // LN):
      asum[pl.ds(q * LN, LN)] = sus[q]
      asq[pl.ds(q * LN, LN)] = sqs[q]

    eb = s * EPT + b * B
    pltpu.async_copy(hb.at[k], h_out.at[c].at[pl.ds(eb, B)], wsem[k])

  def outer(g, _):
    stage(2 * g, 0, 1)
    stage(2 * g + 1, 1, 0)
    return 0

  lax.fori_loop(0, NB // 2, outer, 0)
  wait_wb(0)
  wait_wb(1)
  pltpu.sync_copy(asum, st_out.at[0].at[c].at[pl.ds(s * HH, HH)])
  pltpu.sync_copy(asq, st_out.at[1].at[c].at[pl.ds(s * HH, HH)])


@functools.partial(
    pl.kernel, mesh=_MESH,
    out_type=jax.ShapeDtypeStruct((NC, N, HH), jnp.float32),
    scratch_types=[
        pltpu.VMEM((2, B), jnp.int32),
        pltpu.VMEM((2, B, HH), jnp.float32),
        pltpu.VMEM((ZR, HH), jnp.float32),
        pltpu.VMEM((HH,), jnp.float32),
        pltpu.VMEM((HH,), jnp.float32),
        pltpu.VMEM_SHARED((N, HH), jnp.float32),
        pltpu.SemaphoreType.DMA,
        pltpu.SemaphoreType.DMA,
        pltpu.SemaphoreType.DMA,
        pltpu.SemaphoreType.DMA,
    ])
def _sc_pass_b(h_h, dst_h, scale_h, shift_h, s_out,
               idx_d, hb, zbuf, sc_v, sh_v, acc,
               hsem0, hsem1, ssem0, ssem1):
  c = lax.axis_index("c")
  s = lax.axis_index("s")
  hsem = (hsem0, hsem1)
  ssem = (ssem0, ssem1)

  def zrow(r, _):
    for q in range(HH // LN):
      zbuf[r, pl.ds(q * LN, LN)] = jnp.zeros((LN,), jnp.float32)
    return 0

  lax.fori_loop(0, ZR, zrow, 0)
  for j in range(RPT // ZR):
    pltpu.sync_copy(zbuf, acc.at[pl.ds(s * RPT + j * ZR, ZR)])

  @pl.when(s == NS - 1)
  def _():
    pltpu.sync_copy(zbuf.at[pl.ds(0, RTAIL)], acc.at[pl.ds(NS * RPT, RTAIL)])

  pltpu.sync_copy(scale_h.at[c], sc_v)
  pltpu.sync_copy(shift_h.at[c], sh_v)
  plsc.subcore_barrier()

  def issue_load(b, k):
    eb = s * EPT + b * B
    pltpu.async_copy(dst_h.at[pl.ds(eb, B)], idx_d.at[k], hsem[k])
    pltpu.async_copy(h_h.at[c].at[pl.ds(eb, B)], hb.at[k], hsem[k])

  def wait_load(k):
    pltpu.make_async_copy(dst_h.at[pl.ds(0, B)], idx_d.at[k],
                          hsem[k]).wait()
    pltpu.make_async_copy(h_h.at[c].at[pl.ds(0, B)], hb.at[k],
                          hsem[k]).wait()

  def wait_scatter(k):
    pltpu.make_async_copy(h_h.at[c].at[pl.ds(0, B)], hb.at[k],
                          ssem[k]).wait()

  issue_load(0, 0)

  def stage(b, k, nk):
    @pl.when(b + 1 < NB)
    def _():
      @pl.when(b >= 1)
      def _():
        wait_scatter(nk)

      issue_load(b + 1, nk)

    wait_load(k)

    for q in range(HH // LN):
      sl = pl.ds(q * LN, LN)

      def row_body(r4, _):
        for u in range(4):
          r = 4 * r4 + u
          v = hb[k, r, sl] * sc_v[sl] + sh_v[sl]
          hb[k, r, sl] = jnp.maximum(v, 0.2 * v)
        return 0

      lax.fori_loop(0, B // 4, row_body, 0)

    pltpu.async_copy(hb.at[k], acc.at[idx_d.at[k]], ssem[k], add=True)

  def outer(g, _):
    stage(2 * g, 0, 1)
    stage(2 * g + 1, 1, 0)
    return 0

  lax.fori_loop(0, NB // 2, outer, 0)
  wait_scatter(0)
  wait_scatter(1)
  plsc.subcore_barrier()
  pltpu.sync_copy(acc.at[pl.ds(s * RPT, RPT)],
                  s_out.at[c].at[pl.ds(s * RPT, RPT)])

  @pl.when(s == NS - 1)
  def _():
    pltpu.sync_copy(acc.at[pl.ds(NS * RPT, RTAIL)],
                    s_out.at[c].at[pl.ds(NS * RPT, RTAIL)])


EPC = E // NC
EPCT = EPC // NS
NBC = EPCT // B


@functools.partial(
    pl.kernel, mesh=_MESH,
    out_type=jax.ShapeDtypeStruct((NC, N, HH), jnp.float32),
    scratch_types=[
        pltpu.VMEM((B,), jnp.int32),
        pltpu.VMEM((B, HH), jnp.float32),
        pltpu.VMEM((ZR, HH), jnp.float32),
        pltpu.VMEM_SHARED((N, HH), jnp.float32),
    ])
def _sc_count(dst_h, cnt_out, idx_d, ones_b, zbuf, cnt_sh):
  c = lax.axis_index("c")
  s = lax.axis_index("s")

  def zrow(r, _):
    for q in range(HH // LN):
      zbuf[r, pl.ds(q * LN, LN)] = jnp.zeros((LN,), jnp.float32)
    return 0

  lax.fori_loop(0, ZR, zrow, 0)
  for j in range(RPT // ZR):
    pltpu.sync_copy(zbuf, cnt_sh.at[pl.ds(s * RPT + j * ZR, ZR)])

  @pl.when(s == NS - 1)
  def _():
    pltpu.sync_copy(zbuf.at[pl.ds(0, RTAIL)],
                    cnt_sh.at[pl.ds(NS * RPT, RTAIL)])

  def orow(r, _):
    for q in range(HH // LN):
      ones_b[r, pl.ds(q * LN, LN)] = jnp.full((LN,), 1.0, jnp.float32)
    return 0

  lax.fori_loop(0, B, orow, 0)
  plsc.subcore_barrier()

  def batch_body(b, _):
    eb = c * EPC + s * EPCT + b * B
    pltpu.sync_copy(dst_h.at[pl.ds(eb, B)], idx_d)
    pltpu.sync_copy(ones_b, cnt_sh.at[idx_d], add=True)
    return 0

  lax.fori_loop(0, NBC, batch_body, 0)
  plsc.subcore_barrier()
  pltpu.sync_copy(cnt_sh.at[pl.ds(s * RPT, RPT)],
                  cnt_out.at[c].at[pl.ds(s * RPT, RPT)])

  @pl.when(s == NS - 1)
  def _():
    pltpu.sync_copy(cnt_sh.at[pl.ds(NS * RPT, RTAIL)],
                    cnt_out.at[c].at[pl.ds(NS * RPT, RTAIL)])


def _layer_sc(PiT, PjT, T, dst, src, g, be):
  h_e, stats = _sc_pass_a(PiT, PjT, T, dst, src)
  scale, shift = _tc_stats(stats, g, be)
  return _sc_pass_b(h_e, dst, scale, shift)


def _layer_sc_cnt(PiT, PjT, T, dst, src, g, be):
  return _layer_sc(PiT, PjT, T, dst, src, g, be)


def kernel(x, edge_index, edge_attr, batch,
           W1a, b1a, g1, be1, W1b, b1b,
           W2a, b2a, g2, be2, W2b, b2b,
           gp, bp, Wmu, bmu, Wlv, blv):
  src = edge_index[0]
  dst = edge_index[1]
  ea8 = jnp.concatenate(
      [edge_attr, jnp.zeros((E, 4), jnp.float32)], axis=1)
  Wae1 = jnp.concatenate([W1a[2 * D:], jnp.zeros((4, H), jnp.float32)], 0)
  Wae2 = jnp.concatenate([W2a[2 * H:], jnp.zeros((4, H), jnp.float32)], 0)

  PiT1, PjT1 = _tc_proj(x, W1a[:D], W1a[D:2 * D])
  T1 = _tc_ebias(ea8, Wae1, b1a.reshape(1, H))
  cnt_tab = _sc_count(dst)
  s1 = _layer_sc(PiT1, PjT1, T1, dst, src, g1, be1)
  h_nodes, PiT2, PjT2 = _tc_update(s1, cnt_tab, W1b, b1b.reshape(1, H),
                                   W2a[:H], W2a[H:2 * H])
  T2 = _tc_ebias(ea8, Wae2, b2a.reshape(1, H))
  s2 = _layer_sc(PiT2, PjT2, T2, dst, src, g2, be2)
  h2 = _tc_update(s2, cnt_tab, W2b, b2b.reshape(1, H))[0]

  pooled = _tc_pool(h2, batch.reshape(NBLK, 1, BN))
  mu, lv = _tc_head(pooled, gp, bp, Wmu, bmu, Wlv, blv)
  return mu, lv, h2
